# Initial kernel scaffold; baseline (speedup 1.0000x reference)
import jax
import jax.numpy as jnp
from jax.experimental import pallas as pl


def kernel(x, salt_idx, pepper_idx):
    raise NotImplementedError("write your pallas kernel here")


# speedup vs baseline: 2.8322x; 2.8322x over previous
import functools

import jax
import jax.numpy as jnp
from jax import lax
from jax.experimental import pallas as pl
from jax.experimental.pallas import tpu as pltpu
from jax.experimental.pallas import tpu_sc as plsc

AMOUNT = 0.01
S_VS_P = 0.5
_SHAPE = (64, 3, 384, 384)
_NUMEL = 64 * 3 * 384 * 384
_NIDX = int(AMOUNT * S_VS_P * _NUMEL)

_NC = 2
_NS = 16
_NW = _NC * _NS
_CHUNK = 128
_KCH = -(-_NIDX // (_NW * _CHUNK))
_PER_W = _KCH * _CHUNK
_PAD = _NW * _PER_W

_ROWS = 27648
_COLS = 1024
_BLK = 1024


def _copy_body(x_ref, o_ref):
    o_ref[...] = x_ref[...]


@jax.jit
def _tc_copy(flat2d):
    return pl.pallas_call(
        _copy_body,
        grid=(_ROWS // _BLK,),
        in_specs=[pl.BlockSpec((_BLK, _COLS), lambda i: (i, 0))],
        out_specs=pl.BlockSpec((_BLK, _COLS), lambda i: (i, 0)),
        out_shape=jax.ShapeDtypeStruct((_ROWS, _COLS), jnp.float32),
    )(flat2d)


def _sc_scatter_body(out_hbm, salt_hbm, pepper_hbm,
                     idx_s, idx_p, val1, val0, sem):
    c = lax.axis_index("c")
    s = lax.axis_index("s")
    wid = s * _NC + c
    pltpu.sync_copy(salt_hbm.at[wid], idx_s)
    pltpu.sync_copy(pepper_hbm.at[wid], idx_p)

    ones = jnp.full((16,), 1.0, jnp.float32)
    zeros = jnp.zeros((16,), jnp.float32)
    for i in range(_CHUNK // 16):
        val1[pl.ds(i * 16, 16)] = ones
        val0[pl.ds(i * 16, 16)] = zeros

    @pl.loop(0, _KCH)
    def _fire(j):
        pltpu.make_async_copy(val1, out_hbm.at[idx_s.at[j]], sem).start()
        pltpu.make_async_copy(val0, out_hbm.at[idx_p.at[j]], sem).start()

    @pl.loop(0, _KCH)
    def _drain(j):
        pltpu.make_async_copy(val1, out_hbm.at[idx_s.at[j]], sem).wait()
        pltpu.make_async_copy(val0, out_hbm.at[idx_p.at[j]], sem).wait()


@functools.cache
def _sc_scatter(interpret=False):
    mesh = plsc.VectorSubcoreMesh(
        core_axis_name="c", subcore_axis_name="s",
        num_cores=_NC, num_subcores=_NS)
    return pl.kernel(
        _sc_scatter_body,
        out_type=(),
        mesh=mesh,
        interpret=interpret,
        scratch_types=[
            pltpu.VMEM((_KCH, _CHUNK), jnp.int32),
            pltpu.VMEM((_KCH, _CHUNK), jnp.int32),
            pltpu.VMEM((_CHUNK,), jnp.float32),
            pltpu.VMEM((_CHUNK,), jnp.float32),
            pltpu.SemaphoreType.DMA,
        ],
    )


def _pad_idx(idx):
    pad_n = _PAD - _NIDX
    return jnp.concatenate([idx, idx[:pad_n]]).reshape(_NW, _KCH, _CHUNK)


def kernel(x, salt_idx, pepper_idx):
    flat = x.reshape(_ROWS, _COLS)
    out = _tc_copy(flat)
    salt_p = _pad_idx(salt_idx)
    pepper_p = _pad_idx(pepper_idx)
    out_ref = jax.new_ref(out.reshape(_NUMEL))
    _sc_scatter()(out_ref, salt_p, pepper_p)
    return out_ref[...].reshape(_SHAPE)


# speedup vs baseline: 3.6317x; 1.2823x over previous
import functools

import jax
import jax.numpy as jnp
from jax import lax
from jax.experimental import pallas as pl
from jax.experimental.pallas import tpu as pltpu
from jax.experimental.pallas import tpu_sc as plsc

AMOUNT = 0.01
S_VS_P = 0.5
_SHAPE = (64, 3, 384, 384)
_NUMEL = 64 * 3 * 384 * 384
_NIDX = int(AMOUNT * S_VS_P * _NUMEL)

_NC = 2
_NS = 16
_NW = _NC * _NS
_CHUNK = 128
_KCH = -(-_NIDX // (_NW * _CHUNK))
_PER_W = _KCH * _CHUNK
_PAD = _NW * _PER_W

_ROWS = 27648
_COLS = 1024
_BLK = 1024


def _copy_body(x_ref, o_ref):
    o_ref[...] = x_ref[...]


@jax.jit
def _tc_copy(flat2d):
    return pl.pallas_call(
        _copy_body,
        grid=(_ROWS // _BLK,),
        in_specs=[pl.BlockSpec((_BLK, _COLS), lambda i: (i, 0))],
        out_specs=pl.BlockSpec((_BLK, _COLS), lambda i: (i, 0)),
        out_shape=jax.ShapeDtypeStruct((_ROWS, _COLS), jnp.float32),
    )(flat2d)


def _sc_scatter_body(out_hbm, salt_hbm, pepper_hbm,
                     idx_s, idx_p, val1, val0, sem):
    c = lax.axis_index("c")
    s = lax.axis_index("s")
    wid = s * _NC + c
    pltpu.sync_copy(salt_hbm.at[wid], idx_s)
    pltpu.sync_copy(pepper_hbm.at[wid], idx_p)

    ones = jnp.full((16,), 1.0, jnp.float32)
    zeros = jnp.zeros((16,), jnp.float32)
    for i in range(_CHUNK // 16):
        val1[pl.ds(i * 16, 16)] = ones
        val0[pl.ds(i * 16, 16)] = zeros

    @pl.loop(0, _KCH)
    def _fire(j):
        pltpu.make_async_copy(val1, out_hbm.at[idx_s.at[j]], sem).start()
        pltpu.make_async_copy(val0, out_hbm.at[idx_p.at[j]], sem).start()

    @pl.loop(0, _KCH)
    def _drain(j):
        pltpu.make_async_copy(val1, out_hbm.at[idx_s.at[j]], sem).wait()
        pltpu.make_async_copy(val0, out_hbm.at[idx_p.at[j]], sem).wait()


@functools.cache
def _sc_scatter(interpret=False):
    mesh = plsc.VectorSubcoreMesh(
        core_axis_name="c", subcore_axis_name="s",
        num_cores=_NC, num_subcores=_NS)
    return pl.kernel(
        _sc_scatter_body,
        out_type=(),
        mesh=mesh,
        interpret=interpret,
        scratch_types=[
            pltpu.VMEM((_KCH, _CHUNK), jnp.int32),
            pltpu.VMEM((_KCH, _CHUNK), jnp.int32),
            pltpu.VMEM((_CHUNK,), jnp.float32),
            pltpu.VMEM((_CHUNK,), jnp.float32),
            pltpu.SemaphoreType.DMA,
        ],
    )


def _pad_idx(idx):
    pad_n = _PAD - _NIDX
    return jnp.concatenate([idx, idx[:pad_n]]).reshape(_NW, _KCH, _CHUNK)


def kernel(x, salt_idx, pepper_idx):
    flat = x.reshape(_NUMEL)
    salt_p = _pad_idx(salt_idx)
    pepper_p = _pad_idx(pepper_idx)
    out_ref = jax.new_ref(flat)
    _sc_scatter()(out_ref, salt_p, pepper_p)
    return out_ref[...].reshape(_SHAPE)
